# baseline (speedup 1.0000x reference)
import jax
import jax.numpy as jnp
from jax.experimental import pallas as pl

_K_CHUNK = 1024


def _rpq_kernel(x_ref, p_ref, cb_ref, out_ref):
    x = x_ref[0]
    p = p_ref[...]
    cb = cb_ref[...]
    N = x.shape[0]
    K = cb.shape[0]

    proj_t = jax.lax.dot_general(
        p, x, (((0,), (1,)), ((), ())),
        preferred_element_type=jnp.float32,
    )

    xnorm = jnp.sqrt(jnp.sum(proj_t * proj_t, axis=0, keepdims=True))
    xn_t = proj_t / jnp.maximum(xnorm, 1e-12)
    xn2 = jnp.sum(xn_t * xn_t, axis=0, keepdims=True)

    cbnorm = jnp.sqrt(jnp.sum(cb * cb, axis=1, keepdims=True))
    cbn = cb / jnp.maximum(cbnorm, 1e-12)
    cb2 = jnp.sum(cbn * cbn, axis=1, keepdims=True)

    run_val = jnp.full((1, N), jnp.inf, dtype=jnp.float32)
    run_idx = jnp.zeros((1, N), dtype=jnp.int32)
    for k0 in range(0, K, _K_CHUNK):
        cbn_c = cbn[k0:k0 + _K_CHUNK]
        cb2_c = cb2[k0:k0 + _K_CHUNK]
        scores = jax.lax.dot_general(
            cbn_c, xn_t, (((1,), (0,)), ((), ())),
            preferred_element_type=jnp.float32,
        )
        d2 = (cb2_c + xn2) - 2.0 * scores
        dist = jnp.sqrt(jnp.maximum(d2, 0.0))
        cmin = jnp.min(dist, axis=0, keepdims=True)
        iota = jax.lax.broadcasted_iota(jnp.int32, dist.shape, 0) + k0
        cidx = jnp.min(jnp.where(dist <= cmin, iota, K), axis=0,
                       keepdims=True)
        take = cmin < run_val
        run_idx = jnp.where(take, cidx, run_idx)
        run_val = jnp.minimum(run_val, cmin)

    out_ref[0] = run_idx


def kernel(x, random_projection, codebook):
    B, N, D = x.shape
    K, E = codebook.shape
    return pl.pallas_call(
        _rpq_kernel,
        grid=(B,),
        in_specs=[
            pl.BlockSpec((1, N, D), lambda b: (b, 0, 0)),
            pl.BlockSpec((D, E), lambda b: (0, 0)),
            pl.BlockSpec((K, E), lambda b: (0, 0)),
        ],
        out_specs=pl.BlockSpec((1, 1, N), lambda b: (b, 0, 0)),
        out_shape=jax.ShapeDtypeStruct((B, 1, N), jnp.int32),
    )(x, random_projection, codebook).reshape(B, N)
